# baseline (speedup 1.0000x reference)
import jax
import jax.numpy as jnp
from jax.experimental import pallas as pl

_TOKENS = 8192
_DM = 4096
_NE = 64
_BM = 512


def _gate_onehot_kernel(x_ref, w_ref, g_ref, out_ref):
    z = jax.lax.dot_general(
        x_ref[...], w_ref[...],
        dimension_numbers=(((1,), (1,)), ((), ())),
        preferred_element_type=jnp.float32,
    )
    z = z + g_ref[...]
    m = jnp.max(z, axis=1, keepdims=True)
    iota = jax.lax.broadcasted_iota(jnp.int32, z.shape, 1)
    cand = jnp.where(z >= m, iota, _NE)
    first = jnp.min(cand, axis=1, keepdims=True)
    out_ref[...] = (iota == first).astype(jnp.float32)


def kernel(x, gate_weights):
    u = jax.random.uniform(jax.random.key(1), (_TOKENS, _NE), dtype=jnp.float32)
    g = -jnp.log(-jnp.log(u + 1e-20) + 1e-20)
    return pl.pallas_call(
        _gate_onehot_kernel,
        grid=(_TOKENS // _BM,),
        in_specs=[
            pl.BlockSpec((_BM, _DM), lambda i: (i, 0)),
            pl.BlockSpec((_NE, _DM), lambda i: (0, 0)),
            pl.BlockSpec((_BM, _NE), lambda i: (i, 0)),
        ],
        out_specs=pl.BlockSpec((_BM, _NE), lambda i: (i, 0)),
        out_shape=jax.ShapeDtypeStruct((_TOKENS, _NE), jnp.float32),
    )(x, gate_weights, g)
